# baseline (speedup 1.0000x reference)
import functools

import jax
import jax.numpy as jnp
from jax import lax
from jax.experimental import pallas as pl
from jax.experimental.pallas import tpu as pltpu
from jax.experimental.pallas import tpu_sc as plsc

N = 10000
E = 320000
NC = 2
NS = 16
CHUNK = 80

_MESH = plsc.VectorSubcoreMesh(core_axis_name="c", subcore_axis_name="s")


def _deg_kernel(dst_hbm, out_hbm, dst_v, dst_w, ones_v, zeros_v, acc_sh,
                isems):
    c = lax.axis_index("c")
    s = lax.axis_index("s")

    @pl.loop(0, 125)
    def _(i):
        zeros_v[pl.ds(i * 16, 16)] = jnp.zeros((16,), jnp.float32)

    @pl.loop(0, 5)
    def _(k):
        pltpu.sync_copy(zeros_v, acc_sh.at[pl.ds(k * 2000, 2000)])

    @pl.loop(0, CHUNK // 16)
    def _(i):
        ones_v[pl.ds(i * 16, 16)] = jnp.ones((16,), jnp.float32)

    plsc.subcore_barrier()

    w = c * NS + s
    epw = E // (NC * NS)
    nchunks = epw // CHUNK

    dbufs = (dst_v, dst_w)

    def load_idx(chunk, b):
        base = w * epw + chunk * CHUNK
        pltpu.async_copy(dst_hbm.at[pl.ds(base, CHUNK)], dbufs[b],
                         isems[b])

    def wait_idx(chunk, b):
        base = w * epw + chunk * CHUNK
        pltpu.make_async_copy(dst_hbm.at[pl.ds(base, CHUNK)],
                              dbufs[b], isems[b]).wait()

    load_idx(0, 0)

    @pl.loop(0, nchunks - 1, step=2)
    def _(i):
        for b in range(2):
            load_idx(i + b + 1, 1 - b)
            wait_idx(i + b, b)
            pltpu.sync_copy(ones_v, acc_sh.at[dbufs[b]], add=True)

    wait_idx(nchunks - 1, 0)
    pltpu.sync_copy(ones_v, acc_sh.at[dbufs[0]], add=True)

    plsc.subcore_barrier()

    pltpu.sync_copy(acc_sh, out_hbm.at[c])


_deg_call = functools.partial(
    pl.kernel,
    out_type=jax.ShapeDtypeStruct((NC, N), jnp.float32),
    mesh=_MESH,
    scratch_types=[
        pltpu.VMEM((CHUNK,), jnp.int32),
        pltpu.VMEM((CHUNK,), jnp.int32),
        pltpu.VMEM((CHUNK,), jnp.float32),
        pltpu.VMEM((2000,), jnp.float32),
        pltpu.VMEM_SHARED((N,), jnp.float32),
        [pltpu.SemaphoreType.DMA] * 2,
    ],
    compiler_params=pltpu.CompilerParams(use_tc_tiling_on_sc=False),
)(_deg_kernel)


_NCHUNKS = E // NS // CHUNK


def _agg_body(dh, table_hbm, src_hbm, dst_hbm, out_hbm,
              src0, dst0, src1, dst1, src2, dst2, rows0, rows1, rows2,
              zeros_v, acc_sh, isems, gsems, ssems):
    c = lax.axis_index("c")
    s = lax.axis_index("s")
    zrows = 25
    npert = N // NS

    @pl.loop(0, zrows)
    def _(i):
        for j in range(dh // 16):
            zeros_v[i, pl.ds(j * 16, 16)] = jnp.zeros((16,), jnp.float32)

    @pl.loop(0, npert // zrows)
    def _(k):
        pltpu.sync_copy(
            zeros_v, acc_sh.at[pl.ds(s * npert + k * zrows, zrows), :])

    plsc.subcore_barrier()

    epw = E // NS
    slots = [(src0, dst0, rows0, isems[0], gsems[0], ssems[0]),
             (src1, dst1, rows1, isems[1], gsems[1], ssems[1]),
             (src2, dst2, rows2, isems[2], gsems[2], ssems[2])]

    def load_idx(chunk, sl):
        base = s * epw + chunk * CHUNK
        pltpu.async_copy(src_hbm.at[pl.ds(base, CHUNK)], sl[0], sl[3])
        pltpu.async_copy(dst_hbm.at[pl.ds(base, CHUNK)], sl[1], sl[3])

    def wait_idx(chunk, sl):
        base = s * epw + chunk * CHUNK
        pltpu.make_async_copy(src_hbm.at[pl.ds(base, CHUNK)], sl[0],
                              sl[3]).wait()
        pltpu.make_async_copy(dst_hbm.at[pl.ds(base, CHUNK)], sl[1],
                              sl[3]).wait()

    def gather(sl):
        pltpu.async_copy(table_hbm.at[c].at[sl[0]], sl[2], sl[4])

    def drain(sl):
        pltpu.make_async_copy(table_hbm.at[c].at[sl[0]], sl[2],
                              sl[4]).wait()

    def scatter(sl):
        pltpu.async_copy(sl[2], acc_sh.at[sl[1]], sl[5], add=True)

    def drain_scatter(sl):
        pltpu.make_async_copy(sl[2], acc_sh.at[sl[1]], sl[5]).wait()

    load_idx(0, slots[0])
    load_idx(1, slots[1])
    wait_idx(0, slots[0])
    gather(slots[0])

    drain(slots[0])
    scatter(slots[0])
    load_idx(2, slots[2])
    wait_idx(1, slots[1])
    gather(slots[1])

    @pl.loop(1, _NCHUNKS - 3, step=3)
    def _(i):
        for b in range(3):
            cur = slots[(1 + b) % 3]
            nxt = slots[(2 + b) % 3]
            pre = slots[b % 3]
            drain(cur)
            scatter(cur)
            drain_scatter(pre)
            load_idx(i + b + 2, pre)
            wait_idx(i + b + 1, nxt)
            gather(nxt)

    drain(slots[1])
    scatter(slots[1])
    drain_scatter(slots[0])
    load_idx(_NCHUNKS - 1, slots[0])
    wait_idx(_NCHUNKS - 2, slots[2])
    gather(slots[2])

    drain(slots[2])
    scatter(slots[2])
    drain_scatter(slots[1])
    wait_idx(_NCHUNKS - 1, slots[0])
    gather(slots[0])

    drain(slots[0])
    scatter(slots[0])
    drain_scatter(slots[2])
    drain_scatter(slots[0])

    plsc.subcore_barrier()

    pltpu.sync_copy(acc_sh.at[pl.ds(s * npert, npert), :],
                    out_hbm.at[c, pl.ds(s * npert, npert), :])


def _make_agg(dh):
    return functools.partial(
        pl.kernel,
        out_type=jax.ShapeDtypeStruct((NC, N, dh), jnp.float32),
        mesh=_MESH,
        scratch_types=[
            pltpu.VMEM((CHUNK,), jnp.int32),
            pltpu.VMEM((CHUNK,), jnp.int32),
            pltpu.VMEM((CHUNK,), jnp.int32),
            pltpu.VMEM((CHUNK,), jnp.int32),
            pltpu.VMEM((CHUNK,), jnp.int32),
            pltpu.VMEM((CHUNK,), jnp.int32),
            pltpu.VMEM((CHUNK, dh), jnp.float32),
            pltpu.VMEM((CHUNK, dh), jnp.float32),
            pltpu.VMEM((CHUNK, dh), jnp.float32),
            pltpu.VMEM((25, dh), jnp.float32),
            pltpu.VMEM_SHARED((N, dh), jnp.float32),
            [pltpu.SemaphoreType.DMA] * 3,
            [pltpu.SemaphoreType.DMA] * 3,
            [pltpu.SemaphoreType.DMA] * 3,
        ],
        compiler_params=pltpu.CompilerParams(use_tc_tiling_on_sc=False),
    )(functools.partial(_agg_body, dh))


_agg64 = _make_agg(64)
_agg128 = _make_agg(128)


_BLK = 2000


def _scale_kernel(deg_ref, x_ref, dinv_ref, xs_ref):
    deg = deg_ref[:, 0] + deg_ref[:, 1] + 1.0
    dinv = lax.rsqrt(deg)
    dinv_ref[...] = dinv[:, None]
    xs = x_ref[...] * dinv[:, None]
    xs_ref[0, :, :] = xs[:, :64]
    xs_ref[1, :, :] = xs[:, 64:]


def _scale_call(deg2, x):
    g = N // _BLK
    return pl.pallas_call(
        _scale_kernel,
        grid=(g,),
        in_specs=[
            pl.BlockSpec((_BLK, NC), lambda i: (i, 0)),
            pl.BlockSpec((_BLK, 128), lambda i: (i, 0)),
        ],
        out_specs=[
            pl.BlockSpec((_BLK, 1), lambda i: (i, 0)),
            pl.BlockSpec((NC, _BLK, 64), lambda i: (0, i, 0)),
        ],
        out_shape=[
            jax.ShapeDtypeStruct((N, 1), jnp.float32),
            jax.ShapeDtypeStruct((NC, N, 64), jnp.float32),
        ],
    )(deg2, x)


def _mm_kernel(relu, post, dh, s_ref, h_ref, dinv_ref, w_ref, o_ref):
    dinv = dinv_ref[...]
    m0 = (s_ref[0] + h_ref[0]) * dinv
    m1 = (s_ref[1] + h_ref[1]) * dinv
    p = (jnp.dot(m0, w_ref[:dh, :], preferred_element_type=jnp.float32)
         + jnp.dot(m1, w_ref[dh:, :], preferred_element_type=jnp.float32))
    if relu:
        p = jnp.maximum(p, 0.0)
    if post:
        o_ref[0, :, :] = p[:, :128] * dinv
        o_ref[1, :, :] = p[:, 128:] * dinv
    else:
        o_ref[...] = p


def _mm_call(s_agg, hprev, dinv, w, relu, post):
    dh = s_agg.shape[2]
    g = N // _BLK
    if post:
        out_shape = jax.ShapeDtypeStruct((NC, N, 128), jnp.float32)
        out_spec = pl.BlockSpec((NC, _BLK, 128), lambda i: (0, i, 0))
    else:
        out_shape = jax.ShapeDtypeStruct((N, 256), jnp.float32)
        out_spec = pl.BlockSpec((_BLK, 256), lambda i: (i, 0))
    return pl.pallas_call(
        functools.partial(_mm_kernel, relu, post, dh),
        grid=(g,),
        in_specs=[
            pl.BlockSpec((NC, _BLK, dh), lambda i: (0, i, 0)),
            pl.BlockSpec((NC, _BLK, dh), lambda i: (0, i, 0)),
            pl.BlockSpec((_BLK, 1), lambda i: (i, 0)),
            pl.BlockSpec((2 * dh, 256), lambda i: (0, 0)),
        ],
        out_specs=out_spec,
        out_shape=out_shape,
    )(s_agg, hprev, dinv, w)


@jax.jit
def kernel(x, edge_index, W1, W2, W3):
    src = edge_index[0].astype(jnp.int32)
    dst = edge_index[1].astype(jnp.int32)

    deg2 = _deg_call(dst)
    dinv, xs = _scale_call(deg2.T, x)

    s1 = _agg64(xs, src, dst)
    h1 = _mm_call(s1, xs, dinv, W1, relu=True, post=True)

    s2 = _agg128(h1, src, dst)
    h2 = _mm_call(s2, h1, dinv, W2, relu=True, post=True)

    s3 = _agg128(h2, src, dst)
    return _mm_call(s3, h2, dinv, W3, relu=False, post=False)


# speedup vs baseline: 1.5807x; 1.5807x over previous
import functools

import jax
import jax.numpy as jnp
from jax import lax
from jax.experimental import pallas as pl
from jax.experimental.pallas import tpu as pltpu
from jax.experimental.pallas import tpu_sc as plsc

N = 10000
E = 320000
NC = 2
NS = 16
CHUNK = 80

_MESH = plsc.VectorSubcoreMesh(core_axis_name="c", subcore_axis_name="s")


def _deg_kernel(dst_hbm, out_hbm, dst_v, dst_w, ones_v, zeros_v, acc_sh,
                isems):
    c = lax.axis_index("c")
    s = lax.axis_index("s")

    @pl.loop(0, 125)
    def _(i):
        zeros_v[pl.ds(i * 16, 16)] = jnp.zeros((16,), jnp.float32)

    @pl.loop(0, 5)
    def _(k):
        pltpu.sync_copy(zeros_v, acc_sh.at[pl.ds(k * 2000, 2000)])

    @pl.loop(0, CHUNK // 16)
    def _(i):
        ones_v[pl.ds(i * 16, 16)] = jnp.ones((16,), jnp.float32)

    plsc.subcore_barrier()

    w = c * NS + s
    epw = E // (NC * NS)
    nchunks = epw // CHUNK

    dbufs = (dst_v, dst_w)

    def load_idx(chunk, b):
        base = w * epw + chunk * CHUNK
        pltpu.async_copy(dst_hbm.at[pl.ds(base, CHUNK)], dbufs[b],
                         isems[b])

    def wait_idx(chunk, b):
        base = w * epw + chunk * CHUNK
        pltpu.make_async_copy(dst_hbm.at[pl.ds(base, CHUNK)],
                              dbufs[b], isems[b]).wait()

    load_idx(0, 0)

    @pl.loop(0, nchunks - 1, step=2)
    def _(i):
        for b in range(2):
            load_idx(i + b + 1, 1 - b)
            wait_idx(i + b, b)
            pltpu.sync_copy(ones_v, acc_sh.at[dbufs[b]], add=True)

    wait_idx(nchunks - 1, 0)
    pltpu.sync_copy(ones_v, acc_sh.at[dbufs[0]], add=True)

    plsc.subcore_barrier()

    pltpu.sync_copy(acc_sh, out_hbm.at[c])


_deg_call = functools.partial(
    pl.kernel,
    out_type=jax.ShapeDtypeStruct((NC, N), jnp.float32),
    mesh=_MESH,
    scratch_types=[
        pltpu.VMEM((CHUNK,), jnp.int32),
        pltpu.VMEM((CHUNK,), jnp.int32),
        pltpu.VMEM((CHUNK,), jnp.float32),
        pltpu.VMEM((2000,), jnp.float32),
        pltpu.VMEM_SHARED((N,), jnp.float32),
        [pltpu.SemaphoreType.DMA] * 2,
    ],
    compiler_params=pltpu.CompilerParams(use_tc_tiling_on_sc=False),
)(_deg_kernel)


_NCHUNKS = E // NS // CHUNK


def _agg_body(dh, table_hbm, src_hbm, dst_hbm, out_hbm,
              src0, dst0, src1, dst1, src2, dst2, src3, dst3,
              rows0, rows1, rows2, rows3, zeros_v, acc_sh, isems, gsems):
    c = lax.axis_index("c")
    s = lax.axis_index("s")
    zrows = 25
    npert = N // NS

    @pl.loop(0, zrows)
    def _(i):
        for j in range(dh // 16):
            zeros_v[i, pl.ds(j * 16, 16)] = jnp.zeros((16,), jnp.float32)

    @pl.loop(0, npert // zrows)
    def _(k):
        pltpu.sync_copy(
            zeros_v, acc_sh.at[pl.ds(s * npert + k * zrows, zrows), :])

    plsc.subcore_barrier()

    epw = E // NS
    slots = [(src0, dst0, rows0, isems[0], gsems[0]),
             (src1, dst1, rows1, isems[1], gsems[1]),
             (src2, dst2, rows2, isems[2], gsems[2]),
             (src3, dst3, rows3, isems[3], gsems[3])]

    def load_idx(chunk, sl):
        base = s * epw + chunk * CHUNK
        pltpu.async_copy(src_hbm.at[pl.ds(base, CHUNK)], sl[0], sl[3])
        pltpu.async_copy(dst_hbm.at[pl.ds(base, CHUNK)], sl[1], sl[3])

    def wait_idx(chunk, sl):
        base = s * epw + chunk * CHUNK
        pltpu.make_async_copy(src_hbm.at[pl.ds(base, CHUNK)], sl[0],
                              sl[3]).wait()
        pltpu.make_async_copy(dst_hbm.at[pl.ds(base, CHUNK)], sl[1],
                              sl[3]).wait()

    def gather(sl):
        pltpu.async_copy(table_hbm.at[c].at[sl[0]], sl[2], sl[4])

    def drain(sl):
        pltpu.make_async_copy(table_hbm.at[c].at[sl[0]], sl[2],
                              sl[4]).wait()

    def scatter(sl):
        pltpu.sync_copy(sl[2], acc_sh.at[sl[1]], add=True)

    load_idx(0, slots[0])
    load_idx(1, slots[1])
    load_idx(2, slots[2])
    wait_idx(0, slots[0])
    gather(slots[0])
    wait_idx(1, slots[1])
    gather(slots[1])

    @pl.loop(0, _NCHUNKS - 6, step=4)
    def _(i):
        for b in range(4):
            cur = slots[b]
            nxt = slots[(b + 2) % 4]
            pre = slots[(b + 3) % 4]
            load_idx(i + b + 3, pre)
            wait_idx(i + b + 2, nxt)
            gather(nxt)
            drain(cur)
            scatter(cur)

    load_idx(247, slots[3])
    wait_idx(246, slots[2])
    gather(slots[2])
    drain(slots[0])
    scatter(slots[0])

    load_idx(248, slots[0])
    wait_idx(247, slots[3])
    gather(slots[3])
    drain(slots[1])
    scatter(slots[1])

    load_idx(249, slots[1])
    wait_idx(248, slots[0])
    gather(slots[0])
    drain(slots[2])
    scatter(slots[2])

    wait_idx(249, slots[1])
    gather(slots[1])
    drain(slots[3])
    scatter(slots[3])

    drain(slots[0])
    scatter(slots[0])

    drain(slots[1])
    scatter(slots[1])

    plsc.subcore_barrier()

    pltpu.sync_copy(acc_sh.at[pl.ds(s * npert, npert), :],
                    out_hbm.at[c, pl.ds(s * npert, npert), :])


def _make_agg(dh):
    return functools.partial(
        pl.kernel,
        out_type=jax.ShapeDtypeStruct((NC, N, dh), jnp.float32),
        mesh=_MESH,
        scratch_types=[
            pltpu.VMEM((CHUNK,), jnp.int32),
            pltpu.VMEM((CHUNK,), jnp.int32),
            pltpu.VMEM((CHUNK,), jnp.int32),
            pltpu.VMEM((CHUNK,), jnp.int32),
            pltpu.VMEM((CHUNK,), jnp.int32),
            pltpu.VMEM((CHUNK,), jnp.int32),
            pltpu.VMEM((CHUNK,), jnp.int32),
            pltpu.VMEM((CHUNK,), jnp.int32),
            pltpu.VMEM((CHUNK, dh), jnp.float32),
            pltpu.VMEM((CHUNK, dh), jnp.float32),
            pltpu.VMEM((CHUNK, dh), jnp.float32),
            pltpu.VMEM((CHUNK, dh), jnp.float32),
            pltpu.VMEM((25, dh), jnp.float32),
            pltpu.VMEM_SHARED((N, dh), jnp.float32),
            [pltpu.SemaphoreType.DMA] * 4,
            [pltpu.SemaphoreType.DMA] * 4,
        ],
        compiler_params=pltpu.CompilerParams(use_tc_tiling_on_sc=False),
    )(functools.partial(_agg_body, dh))


_agg64 = _make_agg(64)
_agg128 = _make_agg(128)


_BLK = 2000


def _scale_kernel(deg_ref, x_ref, dinv_ref, xs_ref):
    deg = deg_ref[:, 0] + deg_ref[:, 1] + 1.0
    dinv = lax.rsqrt(deg)
    dinv_ref[...] = dinv[:, None]
    xs = x_ref[...] * dinv[:, None]
    xs_ref[0, :, :] = xs[:, :64]
    xs_ref[1, :, :] = xs[:, 64:]


def _scale_call(deg2, x):
    g = N // _BLK
    return pl.pallas_call(
        _scale_kernel,
        grid=(g,),
        in_specs=[
            pl.BlockSpec((_BLK, NC), lambda i: (i, 0)),
            pl.BlockSpec((_BLK, 128), lambda i: (i, 0)),
        ],
        out_specs=[
            pl.BlockSpec((_BLK, 1), lambda i: (i, 0)),
            pl.BlockSpec((NC, _BLK, 64), lambda i: (0, i, 0)),
        ],
        out_shape=[
            jax.ShapeDtypeStruct((N, 1), jnp.float32),
            jax.ShapeDtypeStruct((NC, N, 64), jnp.float32),
        ],
    )(deg2, x)


def _mm_kernel(relu, post, dh, s_ref, h_ref, dinv_ref, w_ref, o_ref):
    dinv = dinv_ref[...]
    m0 = (s_ref[0] + h_ref[0]) * dinv
    m1 = (s_ref[1] + h_ref[1]) * dinv
    p = (jnp.dot(m0, w_ref[:dh, :], preferred_element_type=jnp.float32)
         + jnp.dot(m1, w_ref[dh:, :], preferred_element_type=jnp.float32))
    if relu:
        p = jnp.maximum(p, 0.0)
    if post:
        o_ref[0, :, :] = p[:, :128] * dinv
        o_ref[1, :, :] = p[:, 128:] * dinv
    else:
        o_ref[...] = p


def _mm_call(s_agg, hprev, dinv, w, relu, post):
    dh = s_agg.shape[2]
    g = N // _BLK
    if post:
        out_shape = jax.ShapeDtypeStruct((NC, N, 128), jnp.float32)
        out_spec = pl.BlockSpec((NC, _BLK, 128), lambda i: (0, i, 0))
    else:
        out_shape = jax.ShapeDtypeStruct((N, 256), jnp.float32)
        out_spec = pl.BlockSpec((_BLK, 256), lambda i: (i, 0))
    return pl.pallas_call(
        functools.partial(_mm_kernel, relu, post, dh),
        grid=(g,),
        in_specs=[
            pl.BlockSpec((NC, _BLK, dh), lambda i: (0, i, 0)),
            pl.BlockSpec((NC, _BLK, dh), lambda i: (0, i, 0)),
            pl.BlockSpec((_BLK, 1), lambda i: (i, 0)),
            pl.BlockSpec((2 * dh, 256), lambda i: (0, 0)),
        ],
        out_specs=out_spec,
        out_shape=out_shape,
    )(s_agg, hprev, dinv, w)


@jax.jit
def kernel(x, edge_index, W1, W2, W3):
    src = edge_index[0].astype(jnp.int32)
    dst = edge_index[1].astype(jnp.int32)

    deg2 = _deg_call(dst)
    dinv, xs = _scale_call(deg2.T, x)

    s1 = _agg64(xs, src, dst)
    h1 = _mm_call(s1, xs, dinv, W1, relu=True, post=True)

    s2 = _agg128(h1, src, dst)
    h2 = _mm_call(s2, h1, dinv, W2, relu=True, post=True)

    s3 = _agg128(h2, src, dst)
    return _mm_call(s3, h2, dinv, W3, relu=False, post=False)
